# Initial kernel scaffold; baseline (speedup 1.0000x reference)
import jax
import jax.numpy as jnp
from jax.experimental import pallas as pl


def kernel(updates, mask):
    raise NotImplementedError("write your pallas kernel here")


# speedup vs baseline: 5.1386x; 5.1386x over previous
import jax
import jax.numpy as jnp
import numpy as np
from jax import lax
from jax.experimental import pallas as pl
from jax.experimental.pallas import tpu as pltpu
from jax.experimental.pallas import tpu_sc as plsc

B = 4
H = W = 192
C = 96
IN_PER_B = H * W * C
OUT_PER_B = 4 * IN_PER_B
TOTAL_IN = B * IN_PER_B
TOTAL_OUT = B * OUT_PER_B

NC = 2
NS = 16
LANES = 16

CHUNKS_PER_B = 8
CHUNK = OUT_PER_B // CHUNKS_PER_B
ROUNDS_PER_B = CHUNKS_PER_B // NC
N_ROUNDS = B * ROUNDS_PER_B

ZB = 2304
ACC_PAD = NS * ZB
ACC_SIZE = CHUNK + ACC_PAD
ZCOPIES = ACC_SIZE // NS // ZB

SLICE = IN_PER_B // NS
WIN = 3072
N_WIN = SLICE // WIN
BLKS = WIN // 128
SLICE_ROWS = SLICE // 128
ROWS_PER_B = IN_PER_B // 128
FLUSH = CHUNK // NS

_F_THIRD = np.float32(1.0 / 3.0)
_F_BIAS = np.float32(0.1)


def _body(upd_hbm, mask_hbm, out_hbm, mask_buf, val_buf, idx_buf, zero_buf, acc):
    core = lax.axis_index("c")
    s = lax.axis_index("s")
    iota16 = lax.iota(jnp.int32, 16)
    dump0 = jnp.int32(CHUNK) + s * ZB

    z16 = jnp.zeros((16,), jnp.float32)

    def zfill(i, _):
        zero_buf[pl.ds(i * 16, 16)] = z16
        return 0

    lax.fori_loop(0, ZB // 16, zfill, 0)

    def round_body(rnd, _):
        b = rnd // ROUNDS_PER_B
        r = rnd - b * ROUNDS_PER_B
        chunk_id = 2 * r + core
        chunk_base = chunk_id * CHUNK

        def zero_step(i, _):
            pltpu.sync_copy(
                zero_buf, acc.at[pl.ds((s * ZCOPIES + i) * ZB, ZB)])
            return 0

        lax.fori_loop(0, ZCOPIES, zero_step, 0)
        plsc.subcore_barrier()

        win0 = b * IN_PER_B + s * SLICE
        row0 = b * ROWS_PER_B + s * SLICE_ROWS

        def win_body(w, _):
            pltpu.sync_copy(mask_hbm.at[pl.ds(win0 + w * WIN, WIN)], mask_buf)
            pltpu.sync_copy(upd_hbm.at[pl.ds(row0 + w * BLKS, BLKS)], val_buf)

            def blk_body(blk, _):
                off = blk * 128
                for u in range(8):
                    o = off + u * 16
                    m = mask_buf[pl.ds(o, 16)]
                    t = lax.shift_right_arithmetic(m, 5)
                    q = (t.astype(jnp.float32) * _F_THIRD + _F_BIAS
                         ).astype(jnp.int32)
                    cu = lax.rem(32 * blk + 16 * u, 96)
                    rel = q * 96 + (cu - chunk_base)
                    valid = plsc.bitcast(rel, jnp.uint32) < jnp.uint32(CHUNK)
                    dmp = dump0 + lax.rem(o, ZB)
                    idx = jnp.where(valid, rel, dmp) + iota16
                    idx_buf[blk, pl.ds(u * 16, 16)] = idx
                return 0

            lax.fori_loop(0, BLKS, blk_body, 0)

            def sc_body(blk, _):
                pltpu.sync_copy(val_buf.at[blk], acc.at[idx_buf.at[blk]],
                                add=True)
                return 0

            lax.fori_loop(0, BLKS, sc_body, 0)
            return 0

        lax.fori_loop(0, N_WIN, win_body, 0)
        plsc.subcore_barrier()

        out_base = b * OUT_PER_B + chunk_base + s * FLUSH
        pltpu.sync_copy(acc.at[pl.ds(s * FLUSH, FLUSH)],
                        out_hbm.at[pl.ds(out_base, FLUSH)])
        plsc.subcore_barrier()
        return 0

    lax.fori_loop(0, N_ROUNDS, round_body, 0)


def kernel(updates, mask):
    upd2 = updates.reshape(TOTAL_IN // 128, 128)
    mask_flat = mask.astype(jnp.int32).reshape(TOTAL_IN)
    mesh = plsc.VectorSubcoreMesh(core_axis_name="c", subcore_axis_name="s")
    out = pl.kernel(
        _body,
        out_type=jax.ShapeDtypeStruct((TOTAL_OUT,), jnp.float32),
        mesh=mesh,
        scratch_types=[
            pltpu.VMEM((WIN,), jnp.int32),
            pltpu.VMEM((BLKS, 128), jnp.float32),
            pltpu.VMEM((BLKS, 128), jnp.int32),
            pltpu.VMEM((ZB,), jnp.float32),
            pltpu.VMEM_SHARED((ACC_SIZE,), jnp.float32),
        ],
    )(upd2, mask_flat)
    return out.reshape(B, 2 * H, 2 * W, C)


# speedup vs baseline: 7.7539x; 1.5089x over previous
import jax
import jax.numpy as jnp
import numpy as np
from jax import lax
from jax.experimental import pallas as pl
from jax.experimental.pallas import tpu as pltpu
from jax.experimental.pallas import tpu_sc as plsc

B = 4
H = W = 192
C = 96
IN_PER_B = H * W * C
OUT_PER_B = 4 * IN_PER_B
TOTAL_IN = B * IN_PER_B
TOTAL_OUT = B * OUT_PER_B

NC = 2
NS = 16

CHUNKS_PER_B = 8
CHUNK = OUT_PER_B // CHUNKS_PER_B
ROUNDS_PER_B = CHUNKS_PER_B // NC
N_ROUNDS = B * ROUNDS_PER_B

DUMP = 1024
ACC_PAD = NS * DUMP
ACC_SIZE = CHUNK + ACC_PAD
ZSLICE = ACC_SIZE // NS

SLICE = IN_PER_B // NS
WIN = 3072
N_WIN = SLICE // WIN
N_PAIR = N_WIN // 2
BLKS = WIN // 128
SLICE_ROWS = SLICE // 128
ROWS_PER_B = IN_PER_B // 128
FLUSH = CHUNK // NS

_F_THIRD = np.float32(1.0 / 3.0)
_F_BIAS = np.float32(0.1)


def _body(upd_hbm, mask_hbm, zero_hbm, out_hbm,
          mask_buf, val_buf, idx_buf, acc, sem_ev, sem_od, sem_sc):
    core = lax.axis_index("c")
    s = lax.axis_index("s")
    iota16 = lax.iota(jnp.int32, 16)
    dump0 = jnp.int32(CHUNK) + s * DUMP

    def round_body(rnd, _):
        b = rnd // ROUNDS_PER_B
        r = rnd - b * ROUNDS_PER_B
        chunk_base = (2 * r + core) * CHUNK
        cbs = [jnp.int32(cu) - chunk_base for cu in range(0, 96, 16)]

        pltpu.sync_copy(zero_hbm.at[pl.ds(s * ZSLICE, ZSLICE)],
                        acc.at[pl.ds(s * ZSLICE, ZSLICE)])
        plsc.subcore_barrier()

        win0 = b * IN_PER_B + s * SLICE
        row0 = b * ROWS_PER_B + s * SLICE_ROWS

        def in_copies(slot, w, sem):
            return (
                pltpu.make_async_copy(
                    mask_hbm.at[pl.ds(win0 + w * WIN, WIN)],
                    mask_buf.at[slot], sem),
                pltpu.make_async_copy(
                    upd_hbm.at[pl.ds(row0 + w * BLKS, BLKS)],
                    val_buf.at[slot], sem),
            )

        def fire_loads(slot, w, sem):
            for d in in_copies(slot, w, sem):
                d.start()

        def wait_loads(slot, w, sem):
            for d in in_copies(slot, w, sem):
                d.wait()

        def fire_scatters(slot):
            for blk in range(BLKS):
                pltpu.async_copy(val_buf.at[slot, blk],
                                 acc.at[idx_buf.at[slot, blk]],
                                 sem_sc, add=True)

        def drain_scatters(slot):
            for blk in range(BLKS):
                pltpu.make_async_copy(val_buf.at[slot, blk],
                                      acc.at[idx_buf.at[slot, blk]],
                                      sem_sc).wait()

        def compute(slot):
            def g_body(g, _):
                base_o = g * 384
                dbase = dump0 + lax.rem(base_o, 512)
                for t in range(3):
                    blk = g * 3 + t
                    for u in range(8):
                        so = t * 128 + u * 16
                        m = mask_buf[slot, pl.ds(base_o + so, 16)]
                        t2 = lax.shift_right_arithmetic(m, 5)
                        q = (t2.astype(jnp.float32) * _F_THIRD + _F_BIAS
                             ).astype(jnp.int32)
                        cb = cbs[((32 * t + 16 * u) % 96) // 16]
                        rel = q * 96 + cb
                        valid = (plsc.bitcast(rel, jnp.uint32)
                                 < jnp.uint32(CHUNK))
                        idx = jnp.where(valid, rel, dbase + so) + iota16
                        idx_buf[slot, blk, pl.ds(u * 16, 16)] = idx
                return 0

            lax.fori_loop(0, 8, g_body, 0)

        fire_loads(0, 0, sem_ev)

        def pair_body(p, _):
            w0 = 2 * p
            w1 = w0 + 1

            @pl.when(p > 0)
            def _():
                drain_scatters(1)
            fire_loads(1, w1, sem_od)
            wait_loads(0, w0, sem_ev)
            compute(0)
            fire_scatters(0)

            wait_loads(1, w1, sem_od)
            drain_scatters(0)

            @pl.when(p < N_PAIR - 1)
            def _():
                fire_loads(0, w0 + 2, sem_ev)
            compute(1)
            fire_scatters(1)
            return 0

        lax.fori_loop(0, N_PAIR, pair_body, 0)
        drain_scatters(1)
        plsc.subcore_barrier()

        out_base = b * OUT_PER_B + chunk_base + s * FLUSH
        pltpu.sync_copy(acc.at[pl.ds(s * FLUSH, FLUSH)],
                        out_hbm.at[pl.ds(out_base, FLUSH)])
        plsc.subcore_barrier()
        return 0

    lax.fori_loop(0, N_ROUNDS, round_body, 0)


def kernel(updates, mask):
    upd2 = updates.reshape(TOTAL_IN // 128, 128)
    mask_flat = mask.astype(jnp.int32).reshape(TOTAL_IN)
    zeros = jnp.zeros((ACC_SIZE,), jnp.float32)
    mesh = plsc.VectorSubcoreMesh(core_axis_name="c", subcore_axis_name="s")
    out = pl.kernel(
        _body,
        out_type=jax.ShapeDtypeStruct((TOTAL_OUT,), jnp.float32),
        mesh=mesh,
        scratch_types=[
            pltpu.VMEM((2, WIN), jnp.int32),
            pltpu.VMEM((2, BLKS, 128), jnp.float32),
            pltpu.VMEM((2, BLKS, 128), jnp.int32),
            pltpu.VMEM_SHARED((ACC_SIZE,), jnp.float32),
            pltpu.SemaphoreType.DMA,
            pltpu.SemaphoreType.DMA,
            pltpu.SemaphoreType.DMA,
        ],
    )(upd2, mask_flat, zeros)
    return out.reshape(B, 2 * H, 2 * W, C)


# speedup vs baseline: 7.9830x; 1.0295x over previous
import jax
import jax.numpy as jnp
import numpy as np
from jax import lax
from jax.experimental import pallas as pl
from jax.experimental.pallas import tpu as pltpu
from jax.experimental.pallas import tpu_sc as plsc

B = 4
H = W = 192
C = 96
IN_PER_B = H * W * C
OUT_PER_B = 4 * IN_PER_B
TOTAL_IN = B * IN_PER_B
TOTAL_OUT = B * OUT_PER_B

NC = 2
NS = 16

CHUNKS_PER_B = 8
CHUNK = OUT_PER_B // CHUNKS_PER_B
ROUNDS_PER_B = CHUNKS_PER_B // NC
N_ROUNDS = B * ROUNDS_PER_B

DUMP = 1024
ACC_PAD = NS * DUMP
ACC_SIZE = CHUNK + ACC_PAD
ZSLICE = ACC_SIZE // NS

SLICE = IN_PER_B // NS
WIN = 3072
N_WIN = SLICE // WIN
N_PAIR = N_WIN // 2
BLKS = WIN // 128
SLICE_ROWS = SLICE // 128
ROWS_PER_B = IN_PER_B // 128
FLUSH = CHUNK // NS

_F_THIRD = np.float32(1.0 / 3.0)
_F_BIAS = np.float32(0.1)


def _body(upd_hbm, mask_hbm, zero_hbm, out_hbm,
          mask_a, mask_b, val_a, val_b, idx_a, idx_b, acc,
          sem_ev, sem_od, sem_sc):
    mask_bufs = (mask_a, mask_b)
    val_bufs = (val_a, val_b)
    idx_bufs = (idx_a, idx_b)
    core = lax.axis_index("c")
    s = lax.axis_index("s")
    iota16 = lax.iota(jnp.int32, 16)
    dump0 = jnp.int32(CHUNK) + s * DUMP

    def round_body(rnd, _):
        b = rnd // ROUNDS_PER_B
        r = rnd - b * ROUNDS_PER_B
        chunk_base = (2 * r + core) * CHUNK
        cbs = [jnp.int32(cu) - chunk_base for cu in range(0, 96, 16)]

        pltpu.sync_copy(zero_hbm.at[pl.ds(s * ZSLICE, ZSLICE)],
                        acc.at[pl.ds(s * ZSLICE, ZSLICE)])
        plsc.subcore_barrier()

        win0 = b * IN_PER_B + s * SLICE

        def in_copies(slot, w, sem):
            return (
                pltpu.make_async_copy(
                    mask_hbm.at[pl.ds(win0 + w * WIN, WIN)],
                    mask_bufs[slot], sem),
                pltpu.make_async_copy(
                    upd_hbm.at[pl.ds(win0 + w * WIN, WIN)],
                    val_bufs[slot], sem),
            )

        def fire_loads(slot, w, sem):
            for d in in_copies(slot, w, sem):
                d.start()

        def wait_loads(slot, w, sem):
            for d in in_copies(slot, w, sem):
                d.wait()

        def fire_scatters(slot):
            pltpu.async_copy(val_bufs[slot],
                             acc.at[idx_bufs[slot]],
                             sem_sc, add=True)

        def drain_scatters(slot):
            pltpu.make_async_copy(val_bufs[slot],
                                  acc.at[idx_bufs[slot]],
                                  sem_sc).wait()


        def compute(slot):
            def g_body(g, _):
                base_o = g * 384
                dbase = dump0 + lax.rem(base_o, 512)
                for t in range(3):
                    blk = g * 3 + t
                    for u in range(8):
                        so = t * 128 + u * 16
                        m = mask_bufs[slot][pl.ds(base_o + so, 16)]
                        t2 = lax.shift_right_arithmetic(m, 5)
                        q = (t2.astype(jnp.float32) * _F_THIRD + _F_BIAS
                             ).astype(jnp.int32)
                        cb = cbs[((32 * t + 16 * u) % 96) // 16]
                        rel = q * 96 + cb
                        valid = (plsc.bitcast(rel, jnp.uint32)
                                 < jnp.uint32(CHUNK))
                        idx = jnp.where(valid, rel, dbase + so) + iota16
                        idx_bufs[slot][pl.ds(base_o + so, 16)] = idx
                return 0

            lax.fori_loop(0, 8, g_body, 0)

        fire_loads(0, 0, sem_ev)

        def pair_body(p, _):
            w0 = 2 * p
            w1 = w0 + 1

            @pl.when(p > 0)
            def _():
                drain_scatters(1)
            fire_loads(1, w1, sem_od)
            wait_loads(0, w0, sem_ev)
            compute(0)
            fire_scatters(0)

            wait_loads(1, w1, sem_od)
            drain_scatters(0)

            @pl.when(p < N_PAIR - 1)
            def _():
                fire_loads(0, w0 + 2, sem_ev)
            compute(1)
            fire_scatters(1)
            return 0

        lax.fori_loop(0, N_PAIR, pair_body, 0)
        drain_scatters(1)
        plsc.subcore_barrier()

        out_base = b * OUT_PER_B + chunk_base + s * FLUSH
        pltpu.sync_copy(acc.at[pl.ds(s * FLUSH, FLUSH)],
                        out_hbm.at[pl.ds(out_base, FLUSH)])
        plsc.subcore_barrier()
        return 0

    lax.fori_loop(0, N_ROUNDS, round_body, 0)


def kernel(updates, mask):
    upd_flat = updates.reshape(TOTAL_IN)
    mask_flat = mask.astype(jnp.int32).reshape(TOTAL_IN)
    zeros = jnp.zeros((ACC_SIZE,), jnp.float32)
    mesh = plsc.VectorSubcoreMesh(core_axis_name="c", subcore_axis_name="s")
    out = pl.kernel(
        _body,
        out_type=jax.ShapeDtypeStruct((TOTAL_OUT,), jnp.float32),
        mesh=mesh,
        scratch_types=[
            pltpu.VMEM((WIN,), jnp.int32),
            pltpu.VMEM((WIN,), jnp.int32),
            pltpu.VMEM((WIN,), jnp.float32),
            pltpu.VMEM((WIN,), jnp.float32),
            pltpu.VMEM((WIN,), jnp.int32),
            pltpu.VMEM((WIN,), jnp.int32),
            pltpu.VMEM_SHARED((ACC_SIZE,), jnp.float32),
            pltpu.SemaphoreType.DMA,
            pltpu.SemaphoreType.DMA,
            pltpu.SemaphoreType.DMA,
        ],
    )(upd_flat, mask_flat, zeros)
    return out.reshape(B, 2 * H, 2 * W, C)


# speedup vs baseline: 12.7445x; 1.5965x over previous
import jax
import jax.numpy as jnp
import numpy as np
from jax import lax
from jax.experimental import pallas as pl
from jax.experimental.pallas import tpu as pltpu
from jax.experimental.pallas import tpu_sc as plsc

B = 4
H = W = 192
C = 96
IN_PER_B = H * W * C
OUT_PER_B = 4 * IN_PER_B
TOTAL_IN = B * IN_PER_B
TOTAL_OUT = B * OUT_PER_B

NC = 2
NS = 16

CHUNKS_PER_B = 8
CHUNK = OUT_PER_B // CHUNKS_PER_B
ROUNDS_PER_B = CHUNKS_PER_B // NC
N_ROUNDS = B * ROUNDS_PER_B

DUMP = 1024
ACC_PAD = NS * DUMP
ACC_SIZE = CHUNK + ACC_PAD
ZSLICE = ACC_SIZE // NS

SLICE = IN_PER_B // NS
WIN = 3072
N_WIN = SLICE // WIN
N_PAIR = N_WIN // 2
BLKS = WIN // 128
SLICE_ROWS = SLICE // 128
ROWS_PER_B = IN_PER_B // 128
FLUSH = CHUNK // NS

_F_THIRD = np.float32(1.0 / 3.0)
_F_BIAS = np.float32(0.1)


def _body(upd_hbm, mask_hbm, zero_hbm, out_hbm,
          mask_a, mask_b, val_a, val_b, idx_a, idx_b, acc,
          sem_ev, sem_od, sem_sc):
    mask_bufs = (mask_a, mask_b)
    val_bufs = (val_a, val_b)
    idx_bufs = (idx_a, idx_b)
    core = lax.axis_index("c")
    s = lax.axis_index("s")
    iota16 = lax.iota(jnp.int32, 16)
    dump0 = jnp.int32(CHUNK) + s * DUMP
    dmpv = [dump0 + (16 * k) + iota16 for k in range(6)]

    def round_body(rnd, _):
        b = rnd // ROUNDS_PER_B
        r = rnd - b * ROUNDS_PER_B
        chunk_base = (2 * r + core) * CHUNK
        cbv = [iota16 + (jnp.int32(cu) - chunk_base)
               for cu in range(0, 96, 16)]

        pltpu.sync_copy(zero_hbm.at[pl.ds(s * ZSLICE, ZSLICE)],
                        acc.at[pl.ds(s * ZSLICE, ZSLICE)])
        plsc.subcore_barrier()

        win0 = b * IN_PER_B + s * SLICE

        def in_copies(slot, w, sem):
            return (
                pltpu.make_async_copy(
                    mask_hbm.at[pl.ds(win0 + w * WIN, WIN)],
                    mask_bufs[slot], sem),
                pltpu.make_async_copy(
                    upd_hbm.at[pl.ds(win0 + w * WIN, WIN)],
                    val_bufs[slot], sem),
            )

        def fire_loads(slot, w, sem):
            for d in in_copies(slot, w, sem):
                d.start()

        def wait_loads(slot, w, sem):
            for d in in_copies(slot, w, sem):
                d.wait()

        def fire_scatters(slot):
            pltpu.async_copy(val_bufs[slot],
                             acc.at[idx_bufs[slot]],
                             sem_sc, add=True)

        def drain_scatters(slot):
            pltpu.make_async_copy(val_bufs[slot],
                                  acc.at[idx_bufs[slot]],
                                  sem_sc).wait()


        def compute(slot):
            mb = mask_bufs[slot]
            ib = idx_bufs[slot]

            @plsc.parallel_loop(0, WIN, 384)
            def g_body(o0):
                for t in range(3):
                    for u in range(8):
                        so = t * 128 + u * 16
                        k = ((32 * t + 16 * u) % 96) // 16
                        m = mb[pl.ds(o0 + so, 16)]
                        t2 = lax.shift_right_arithmetic(m, 5)
                        q = (t2.astype(jnp.float32) * _F_THIRD + _F_BIAS
                             ).astype(jnp.int32)
                        rel = q * 96 + cbv[k]
                        valid = (plsc.bitcast(rel, jnp.uint32)
                                 < jnp.uint32(CHUNK))
                        ib[pl.ds(o0 + so, 16)] = jnp.where(valid, rel,
                                                           dmpv[k])

        fire_loads(0, 0, sem_ev)

        def pair_body(p, _):
            w0 = 2 * p
            w1 = w0 + 1

            @pl.when(p > 0)
            def _():
                drain_scatters(1)
            fire_loads(1, w1, sem_od)
            wait_loads(0, w0, sem_ev)
            compute(0)
            fire_scatters(0)

            wait_loads(1, w1, sem_od)
            drain_scatters(0)

            @pl.when(p < N_PAIR - 1)
            def _():
                fire_loads(0, w0 + 2, sem_ev)
            compute(1)
            fire_scatters(1)
            return 0

        lax.fori_loop(0, N_PAIR, pair_body, 0)
        drain_scatters(1)
        plsc.subcore_barrier()

        out_base = b * OUT_PER_B + chunk_base + s * FLUSH
        pltpu.sync_copy(acc.at[pl.ds(s * FLUSH, FLUSH)],
                        out_hbm.at[pl.ds(out_base, FLUSH)])
        plsc.subcore_barrier()
        return 0

    lax.fori_loop(0, N_ROUNDS, round_body, 0)


def kernel(updates, mask):
    upd_flat = updates.reshape(TOTAL_IN)
    mask_flat = mask.astype(jnp.int32).reshape(TOTAL_IN)
    zeros = jnp.zeros((ACC_SIZE,), jnp.float32)
    mesh = plsc.VectorSubcoreMesh(core_axis_name="c", subcore_axis_name="s")
    out = pl.kernel(
        _body,
        out_type=jax.ShapeDtypeStruct((TOTAL_OUT,), jnp.float32),
        mesh=mesh,
        scratch_types=[
            pltpu.VMEM((WIN,), jnp.int32),
            pltpu.VMEM((WIN,), jnp.int32),
            pltpu.VMEM((WIN,), jnp.float32),
            pltpu.VMEM((WIN,), jnp.float32),
            pltpu.VMEM((WIN,), jnp.int32),
            pltpu.VMEM((WIN,), jnp.int32),
            pltpu.VMEM_SHARED((ACC_SIZE,), jnp.float32),
            pltpu.SemaphoreType.DMA,
            pltpu.SemaphoreType.DMA,
            pltpu.SemaphoreType.DMA,
        ],
    )(upd_flat, mask_flat, zeros)
    return out.reshape(B, 2 * H, 2 * W, C)


# speedup vs baseline: 15.4694x; 1.2138x over previous
import jax
import jax.numpy as jnp
import numpy as np
from jax import lax
from jax.experimental import pallas as pl
from jax.experimental.pallas import tpu as pltpu
from jax.experimental.pallas import tpu_sc as plsc

B = 4
H = W = 192
C = 96
IN_PER_B = H * W * C
OUT_PER_B = 4 * IN_PER_B
TOTAL_IN = B * IN_PER_B
TOTAL_OUT = B * OUT_PER_B

NC = 2
NS = 16

CHUNKS_PER_B = 8
CHUNK = OUT_PER_B // CHUNKS_PER_B
ROUNDS_PER_B = CHUNKS_PER_B // NC
N_ROUNDS = B * ROUNDS_PER_B

DUMP = 1024
ACC_PAD = NS * DUMP
ACC_SIZE = CHUNK + ACC_PAD
ZSLICE = ACC_SIZE // NS

SLICE = IN_PER_B // NS
WIN = 3072
N_WIN = SLICE // WIN
N_PAIR = N_WIN // 2
FLUSH = CHUNK // NS

_F_THIRD = np.float32(1.0 / 3.0)
_F_BIAS = np.float32(0.1)


def _body(upd_hbm, mask_hbm, zero_hbm, out_hbm,
          mask_a, mask_b, val_a, val_b, idx_a, idx_b, acc,
          sem_ev, sem_od, sem_sc):
    mask_bufs = (mask_a, mask_b)
    val_bufs = (val_a, val_b)
    idx_bufs = (idx_a, idx_b)
    core = lax.axis_index("c")
    s = lax.axis_index("s")
    iota16 = lax.iota(jnp.int32, 16)
    dump0 = jnp.int32(CHUNK) + s * DUMP
    dmpv = [dump0 + (16 * k) + iota16 for k in range(6)]

    def round_body(rnd, _):
        b = rnd // ROUNDS_PER_B
        r = rnd - b * ROUNDS_PER_B
        chunk_base = (2 * r + core) * CHUNK
        cbv = [iota16 + (jnp.int32(cu) - chunk_base)
               for cu in range(0, 96, 16)]

        pltpu.sync_copy(zero_hbm.at[pl.ds(s * ZSLICE, ZSLICE)],
                        acc.at[pl.ds(s * ZSLICE, ZSLICE)])
        plsc.subcore_barrier()

        win0 = b * IN_PER_B + s * SLICE

        def in_copies(slot, w, sem):
            return (
                pltpu.make_async_copy(
                    mask_hbm.at[pl.ds(win0 + w * WIN, WIN)],
                    mask_bufs[slot], sem),
                pltpu.make_async_copy(
                    upd_hbm.at[pl.ds(win0 + w * WIN, WIN)],
                    val_bufs[slot], sem),
            )

        def fire_loads(slot, w, sem):
            for d in in_copies(slot, w, sem):
                d.start()

        def wait_loads(slot, w, sem):
            for d in in_copies(slot, w, sem):
                d.wait()

        def fire_scatters(slot):
            pltpu.async_copy(val_bufs[slot],
                             acc.at[idx_bufs[slot]],
                             sem_sc, add=True)

        def drain_scatters(slot):
            pltpu.make_async_copy(val_bufs[slot],
                                  acc.at[idx_bufs[slot]],
                                  sem_sc).wait()

        def compute(slot):
            mb = mask_bufs[slot]
            vb = val_bufs[slot]
            ib = idx_bufs[slot]

            @plsc.parallel_loop(0, WIN, 96)
            def g_body(o0):
                for u in range(6):
                    o = o0 + 16 * u
                    m = mb[pl.ds(o, 16)]
                    t2 = lax.shift_right_arithmetic(m, 5)
                    q = (t2.astype(jnp.float32) * _F_THIRD + _F_BIAS
                         ).astype(jnp.int32)
                    rel = q * 96 + cbv[u]
                    valid = (plsc.bitcast(rel, jnp.uint32)
                             < jnp.uint32(CHUNK))
                    ib[pl.ds(o, 16)] = jnp.where(valid, rel,
                                                 rel & jnp.int32(0xFFFFF))
                    vv = vb[pl.ds(o, 16)]
                    vb[pl.ds(o, 16)] = jnp.where(valid, vv,
                                                 jnp.float32(0.0))

        fire_loads(0, 0, sem_ev)

        def pair_body(p, _):
            w0 = 2 * p
            w1 = w0 + 1

            @pl.when(p > 0)
            def _():
                drain_scatters(1)
            fire_loads(1, w1, sem_od)
            wait_loads(0, w0, sem_ev)
            compute(0)
            fire_scatters(0)

            wait_loads(1, w1, sem_od)
            drain_scatters(0)

            @pl.when(p < N_PAIR - 1)
            def _():
                fire_loads(0, w0 + 2, sem_ev)
            compute(1)
            fire_scatters(1)
            return 0

        lax.fori_loop(0, N_PAIR, pair_body, 0)
        drain_scatters(1)
        plsc.subcore_barrier()

        out_base = b * OUT_PER_B + chunk_base + s * FLUSH
        pltpu.sync_copy(acc.at[pl.ds(s * FLUSH, FLUSH)],
                        out_hbm.at[pl.ds(out_base, FLUSH)])
        plsc.subcore_barrier()
        return 0

    lax.fori_loop(0, N_ROUNDS, round_body, 0)


def kernel(updates, mask):
    upd_flat = updates.reshape(TOTAL_IN)
    mask_flat = mask.astype(jnp.int32).reshape(TOTAL_IN)
    zeros = jnp.zeros((ACC_SIZE,), jnp.float32)
    mesh = plsc.VectorSubcoreMesh(core_axis_name="c", subcore_axis_name="s")
    out = pl.kernel(
        _body,
        out_type=jax.ShapeDtypeStruct((TOTAL_OUT,), jnp.float32),
        mesh=mesh,
        scratch_types=[
            pltpu.VMEM((WIN,), jnp.int32),
            pltpu.VMEM((WIN,), jnp.int32),
            pltpu.VMEM((WIN,), jnp.float32),
            pltpu.VMEM((WIN,), jnp.float32),
            pltpu.VMEM((WIN,), jnp.int32),
            pltpu.VMEM((WIN,), jnp.int32),
            pltpu.VMEM_SHARED((ACC_SIZE,), jnp.float32),
            pltpu.SemaphoreType.DMA,
            pltpu.SemaphoreType.DMA,
            pltpu.SemaphoreType.DMA,
        ],
    )(upd_flat, mask_flat, zeros)
    return out.reshape(B, 2 * H, 2 * W, C)


# speedup vs baseline: 16.0036x; 1.0345x over previous
import jax
import jax.numpy as jnp
import numpy as np
from jax import lax
from jax.experimental import pallas as pl
from jax.experimental.pallas import tpu as pltpu
from jax.experimental.pallas import tpu_sc as plsc

B = 4
H = W = 192
C = 96
IN_PER_B = H * W * C
OUT_PER_B = 4 * IN_PER_B
TOTAL_IN = B * IN_PER_B
TOTAL_OUT = B * OUT_PER_B

NC = 2
NS = 16

CHUNKS_PER_B = 8
CHUNK = OUT_PER_B // CHUNKS_PER_B
ROUNDS_PER_B = CHUNKS_PER_B // NC
N_ROUNDS = B * ROUNDS_PER_B

ACC_SIZE = CHUNK
ZSLICE = ACC_SIZE // NS

SLICE = IN_PER_B // NS
WIN = 3072
N_WIN = SLICE // WIN
N_PAIR = N_WIN // 2
FLUSH = CHUNK // NS

_F_THIRD = np.float32(1.0 / 3.0)
_F_BIAS = np.float32(0.1)


def _body(upd_hbm, mask_hbm, zero_hbm, out_hbm,
          mask_a, mask_b, val_a, val_b, idx_a, idx_b, acc,
          sem_ev, sem_od, sem_sc):
    mask_bufs = (mask_a, mask_b)
    val_bufs = (val_a, val_b)
    idx_bufs = (idx_a, idx_b)
    core = lax.axis_index("c")
    s = lax.axis_index("s")
    iota16 = lax.iota(jnp.int32, 16)

    pltpu.sync_copy(zero_hbm.at[pl.ds(s * ZSLICE, ZSLICE)],
                    acc.at[pl.ds(s * ZSLICE, ZSLICE)])
    plsc.subcore_barrier()

    def round_body(rnd, _):
        b = rnd // ROUNDS_PER_B
        r = rnd - b * ROUNDS_PER_B
        chunk_base = (2 * r + core) * CHUNK
        cbv = [iota16 + (jnp.int32(cu) - chunk_base)
               for cu in range(0, 96, 16)]

        win0 = b * IN_PER_B + s * SLICE

        def in_copies(slot, w, sem):
            return (
                pltpu.make_async_copy(
                    mask_hbm.at[pl.ds(win0 + w * WIN, WIN)],
                    mask_bufs[slot], sem),
                pltpu.make_async_copy(
                    upd_hbm.at[pl.ds(win0 + w * WIN, WIN)],
                    val_bufs[slot], sem),
            )

        def fire_loads(slot, w, sem):
            for d in in_copies(slot, w, sem):
                d.start()

        def wait_loads(slot, w, sem):
            for d in in_copies(slot, w, sem):
                d.wait()

        def fire_scatters(slot):
            pltpu.async_copy(val_bufs[slot],
                             acc.at[idx_bufs[slot]],
                             sem_sc, add=True)

        def drain_scatters(slot):
            pltpu.make_async_copy(val_bufs[slot],
                                  acc.at[idx_bufs[slot]],
                                  sem_sc).wait()

        def compute(slot):
            mb = mask_bufs[slot]
            vb = val_bufs[slot]
            ib = idx_bufs[slot]

            @plsc.parallel_loop(0, WIN, 96)
            def g_body(o0):
                for u in range(6):
                    o = o0 + 16 * u
                    m = mb[pl.ds(o, 16)]
                    t2 = lax.shift_right_arithmetic(m, 5)
                    q = (t2.astype(jnp.float32) * _F_THIRD + _F_BIAS
                         ).astype(jnp.int32)
                    rel = q * 96 + cbv[u]
                    valid = (plsc.bitcast(rel, jnp.uint32)
                             < jnp.uint32(CHUNK))
                    ib[pl.ds(o, 16)] = jnp.where(valid, rel,
                                                 rel & jnp.int32(0xFFFFF))
                    vv = vb[pl.ds(o, 16)]
                    vb[pl.ds(o, 16)] = jnp.where(valid, vv,
                                                 jnp.float32(0.0))

        fire_loads(0, 0, sem_ev)

        def pair_body(p, _):
            w0 = 2 * p
            w1 = w0 + 1

            @pl.when(p > 0)
            def _():
                drain_scatters(1)
            fire_loads(1, w1, sem_od)
            wait_loads(0, w0, sem_ev)
            compute(0)
            fire_scatters(0)

            wait_loads(1, w1, sem_od)
            drain_scatters(0)

            @pl.when(p < N_PAIR - 1)
            def _():
                fire_loads(0, w0 + 2, sem_ev)
            compute(1)
            fire_scatters(1)
            return 0

        lax.fori_loop(0, N_PAIR, pair_body, 0)
        drain_scatters(1)
        plsc.subcore_barrier()

        out_base = b * OUT_PER_B + chunk_base + s * FLUSH
        pltpu.sync_copy(acc.at[pl.ds(s * FLUSH, FLUSH)],
                        out_hbm.at[pl.ds(out_base, FLUSH)])
        pltpu.sync_copy(zero_hbm.at[pl.ds(s * ZSLICE, ZSLICE)],
                        acc.at[pl.ds(s * ZSLICE, ZSLICE)])
        plsc.subcore_barrier()
        return 0

    lax.fori_loop(0, N_ROUNDS, round_body, 0)


def kernel(updates, mask):
    upd_flat = updates.reshape(TOTAL_IN) * np.float32(1.0)
    mask_flat = mask.astype(jnp.int32).reshape(TOTAL_IN) | np.int32(0)
    zeros = jnp.zeros((ACC_SIZE,), jnp.float32)
    mesh = plsc.VectorSubcoreMesh(core_axis_name="c", subcore_axis_name="s")
    out = pl.kernel(
        _body,
        out_type=jax.ShapeDtypeStruct((TOTAL_OUT,), jnp.float32),
        mesh=mesh,
        scratch_types=[
            pltpu.VMEM((WIN,), jnp.int32),
            pltpu.VMEM((WIN,), jnp.int32),
            pltpu.VMEM((WIN,), jnp.float32),
            pltpu.VMEM((WIN,), jnp.float32),
            pltpu.VMEM((WIN,), jnp.int32),
            pltpu.VMEM((WIN,), jnp.int32),
            pltpu.VMEM_SHARED((ACC_SIZE,), jnp.float32),
            pltpu.SemaphoreType.DMA,
            pltpu.SemaphoreType.DMA,
            pltpu.SemaphoreType.DMA,
        ],
    )(upd_flat, mask_flat, zeros)
    return out.reshape(B, 2 * H, 2 * W, C)
